# baseline (speedup 1.0000x reference)
import functools

import jax
import jax.numpy as jnp
from jax import lax
from jax.experimental import pallas as pl
from jax.experimental.pallas import tpu as pltpu
from jax.experimental.pallas import tpu_sc as plsc

NC = 2
NS = 16
L = 16
NW = NC * NS

B = 1024
S = 16
D = 128
E = 128

B_PER_W = B // NW
ROWS_PER_W = B_PER_W * S
GW = 128
N_CH = ROWS_PER_W // GW
NODES_PER_CH = GW // S

_MESH = plsc.VectorSubcoreMesh(core_axis_name="c", subcore_axis_name="s")


def _sc_gather_mean(features, nodes, neigh_flat):
    @functools.partial(
        pl.kernel,
        out_type=(
            jax.ShapeDtypeStruct((B, D), jnp.float32),
            jax.ShapeDtypeStruct((B, D), jnp.float32),
        ),
        mesh=_MESH,
        scratch_types=[
            pltpu.VMEM((ROWS_PER_W,), jnp.int32),
            pltpu.VMEM((B_PER_W,), jnp.int32),
            pltpu.VMEM((ROWS_PER_W, D), jnp.float32),
            pltpu.VMEM((B_PER_W, D), jnp.float32),
            pltpu.VMEM((B_PER_W, D), jnp.float32),
            pltpu.VMEM_SHARED((NS * B_PER_W, D), jnp.float32),
            [pltpu.SemaphoreType.DMA] * N_CH,
            pltpu.SemaphoreType.DMA,
            pltpu.SemaphoreType.DMA,
        ],
    )
    def k(feat_hbm, nodes_hbm, nidx_hbm, self_out, mean_out,
          nidx_v, sidx_v, rows_v, self_v, mean_v, acc_sh,
          gsems, ssem, asem):
        sid = lax.axis_index("s")
        cid = lax.axis_index("c")
        wid = sid * NC + cid
        base = wid * B_PER_W
        sbase = sid * B_PER_W

        pltpu.sync_copy(nidx_hbm.at[pl.ds(wid * ROWS_PER_W, ROWS_PER_W)],
                        nidx_v)
        pltpu.sync_copy(nodes_hbm.at[pl.ds(base, B_PER_W)], sidx_v)

        gcopies = []
        for j in range(N_CH):
            gcopies.append(pltpu.async_copy(
                feat_hbm.at[nidx_v.at[pl.ds(j * GW, GW)]],
                rows_v.at[pl.ds(j * GW, GW)], gsems[j]))
        self_copy = pltpu.async_copy(feat_hbm.at[sidx_v], self_v, ssem)

        @pl.loop(0, B_PER_W)
        def _(n):
            for c in range(D // L):
                mean_v[n, pl.ds(c * L, L)] = jnp.zeros((L,), jnp.float32)

        pltpu.sync_copy(mean_v, acc_sh.at[pl.ds(sbase, B_PER_W)])

        acopies = []
        for j in range(N_CH):
            gcopies[j].wait()
            for n in range(j * NODES_PER_CH, (j + 1) * NODES_PER_CH):
                dst_rows = (sbase + n) + jnp.zeros((L,), jnp.int32)
                acopies.append(pltpu.async_copy(
                    rows_v.at[pl.ds(n * S, S)],
                    acc_sh.at[dst_rows], asem, add=True))
        for c in acopies:
            c.wait()

        pltpu.sync_copy(acc_sh.at[pl.ds(sbase, B_PER_W)], mean_v)

        @pl.loop(0, B_PER_W)
        def _(n):
            for c in range(D // L):
                cs = pl.ds(c * L, L)
                mean_v[n, cs] = mean_v[n, cs] * (1.0 / S)

        self_copy.wait()
        pltpu.sync_copy(self_v, self_out.at[pl.ds(base, B_PER_W)])
        pltpu.sync_copy(mean_v, mean_out.at[pl.ds(base, B_PER_W)])

    return k(features, nodes, neigh_flat)


def _tc_bmm(selff, meanf, weight):
    Bb = 32

    def body(s_ref, m_ref, w_ref, o_ref):
        c = jnp.concatenate([s_ref[...], m_ref[...]], axis=1)
        acc = jax.lax.dot_general(
            c, w_ref[...],
            dimension_numbers=(((1,), (1,)), ((0,), (0,))),
            preferred_element_type=jnp.float32)
        o_ref[...] = jnp.maximum(acc, 0.0)

    return pl.pallas_call(
        body,
        grid=(B // Bb,),
        in_specs=[
            pl.BlockSpec((Bb, D), lambda i: (i, 0)),
            pl.BlockSpec((Bb, D), lambda i: (i, 0)),
            pl.BlockSpec((Bb, 2 * D, E), lambda i: (i, 0, 0)),
        ],
        out_specs=pl.BlockSpec((Bb, E), lambda i: (i, 0)),
        out_shape=jax.ShapeDtypeStruct((B, E), jnp.float32),
    )(selff, meanf, weight)


def kernel(features, nodes, neigh_idx, weight):
    nodes = nodes.astype(jnp.int32)
    neigh_flat = neigh_idx.astype(jnp.int32).reshape(-1)
    selff, meanf = _sc_gather_mean(features, nodes, neigh_flat)
    return _tc_bmm(selff, meanf, weight)


# speedup vs baseline: 1.0179x; 1.0179x over previous
import functools

import jax
import jax.numpy as jnp
from jax import lax
from jax.experimental import pallas as pl
from jax.experimental.pallas import tpu as pltpu
from jax.experimental.pallas import tpu_sc as plsc

NC = 2
NS = 16
L = 16
NW = NC * NS

B = 1024
S = 16
D = 128
E = 128

CB0 = 256
GW = 128

_MESH = plsc.VectorSubcoreMesh(core_axis_name="c", subcore_axis_name="s")


def _sc_gather_mean(features, nodes_c, neigh_flat_c, cb):
    b_per_w = cb // NW
    rows_per_w = b_per_w * S
    n_ch = rows_per_w // GW
    nodes_per_ch = GW // S

    @functools.partial(
        pl.kernel,
        out_type=(
            jax.ShapeDtypeStruct((cb, D), jnp.float32),
            jax.ShapeDtypeStruct((cb, D), jnp.float32),
        ),
        mesh=_MESH,
        scratch_types=[
            pltpu.VMEM((rows_per_w,), jnp.int32),
            pltpu.VMEM((b_per_w,), jnp.int32),
            pltpu.VMEM((rows_per_w, D), jnp.float32),
            pltpu.VMEM((b_per_w, D), jnp.float32),
            pltpu.VMEM((b_per_w, D), jnp.float32),
            pltpu.VMEM_SHARED((NS * b_per_w, D), jnp.float32),
            [pltpu.SemaphoreType.DMA] * n_ch,
            pltpu.SemaphoreType.DMA,
            pltpu.SemaphoreType.DMA,
        ],
    )
    def k(feat_hbm, nodes_hbm, nidx_hbm, self_out, mean_out,
          nidx_v, sidx_v, rows_v, self_v, mean_v, acc_sh,
          gsems, ssem, asem):
        sid = lax.axis_index("s")
        cid = lax.axis_index("c")
        wid = sid * NC + cid
        base = wid * b_per_w
        sbase = sid * b_per_w

        pltpu.sync_copy(nidx_hbm.at[pl.ds(wid * rows_per_w, rows_per_w)],
                        nidx_v)
        pltpu.sync_copy(nodes_hbm.at[pl.ds(base, b_per_w)], sidx_v)

        gcopies = []
        for j in range(n_ch):
            gcopies.append(pltpu.async_copy(
                feat_hbm.at[nidx_v.at[pl.ds(j * GW, GW)]],
                rows_v.at[pl.ds(j * GW, GW)], gsems[j]))
        self_copy = pltpu.async_copy(feat_hbm.at[sidx_v], self_v, ssem)

        @pl.loop(0, b_per_w)
        def _(n):
            for c in range(D // L):
                mean_v[n, pl.ds(c * L, L)] = jnp.zeros((L,), jnp.float32)

        pltpu.sync_copy(mean_v, acc_sh.at[pl.ds(sbase, b_per_w)])

        acopies = []
        for j in range(n_ch):
            gcopies[j].wait()
            for n in range(j * nodes_per_ch, (j + 1) * nodes_per_ch):
                dst_rows = (sbase + n) + jnp.zeros((L,), jnp.int32)
                acopies.append(pltpu.async_copy(
                    rows_v.at[pl.ds(n * S, S)],
                    acc_sh.at[dst_rows], asem, add=True))
        for c in acopies:
            c.wait()

        pltpu.sync_copy(acc_sh.at[pl.ds(sbase, b_per_w)], mean_v)

        @pl.loop(0, b_per_w)
        def _(n):
            for c in range(D // L):
                cs = pl.ds(c * L, L)
                mean_v[n, cs] = mean_v[n, cs] * (1.0 / S)

        self_copy.wait()
        pltpu.sync_copy(self_v, self_out.at[pl.ds(base, b_per_w)])
        pltpu.sync_copy(mean_v, mean_out.at[pl.ds(base, b_per_w)])

    return k(features, nodes_c, neigh_flat_c)


def _tc_bmm(selff, meanf, weight, row0):
    cb = selff.shape[0]
    Bb = 64
    step0 = row0 // Bb

    def body(s_ref, m_ref, w_ref, o_ref):
        c = jnp.concatenate([s_ref[...], m_ref[...]], axis=1)
        acc = jax.lax.dot_general(
            c, w_ref[...],
            dimension_numbers=(((1,), (1,)), ((0,), (0,))),
            preferred_element_type=jnp.float32)
        o_ref[...] = jnp.maximum(acc, 0.0)

    return pl.pallas_call(
        body,
        grid=(cb // Bb,),
        in_specs=[
            pl.BlockSpec((Bb, D), lambda i: (i, 0)),
            pl.BlockSpec((Bb, D), lambda i: (i, 0)),
            pl.BlockSpec((Bb, 2 * D, E), lambda i: (step0 + i, 0, 0)),
        ],
        out_specs=pl.BlockSpec((Bb, E), lambda i: (i, 0)),
        out_shape=jax.ShapeDtypeStruct((cb, E), jnp.float32),
    )(selff, meanf, weight)


def kernel(features, nodes, neigh_idx, weight):
    nodes = nodes.astype(jnp.int32)
    neigh_flat = neigh_idx.astype(jnp.int32).reshape(-1)
    outs = []
    for row0, cb in ((0, CB0), (CB0, B - CB0)):
        nodes_c = lax.slice(nodes, (row0,), (row0 + cb,))
        neigh_c = lax.slice(neigh_flat, (row0 * S,), ((row0 + cb) * S,))
        selff, meanf = _sc_gather_mean(features, nodes_c, neigh_c, cb)
        outs.append(_tc_bmm(selff, meanf, weight, row0))
    return jnp.concatenate(outs, axis=0)


# speedup vs baseline: 1.0622x; 1.0435x over previous
import functools

import jax
import jax.numpy as jnp
from jax import lax
from jax.experimental import pallas as pl
from jax.experimental.pallas import tpu as pltpu
from jax.experimental.pallas import tpu_sc as plsc

NC = 2
NS = 16
L = 16
NW = NC * NS

B = 1024
S = 16
D = 128
E = 128

B_PER_W = B // NW
ROWS_PER_W = B_PER_W * S
GW = 128
N_CH = ROWS_PER_W // GW
NODES_PER_CH = GW // S

_MESH = plsc.VectorSubcoreMesh(core_axis_name="c", subcore_axis_name="s")


def _sc_gather_mean(features, nodes, neigh_flat):
    @functools.partial(
        pl.kernel,
        out_type=(
            jax.ShapeDtypeStruct((B, D), jnp.float32),
            jax.ShapeDtypeStruct((B, D), jnp.float32),
        ),
        mesh=_MESH,
        scratch_types=[
            pltpu.VMEM((ROWS_PER_W,), jnp.int32),
            pltpu.VMEM((B_PER_W,), jnp.int32),
            pltpu.VMEM((ROWS_PER_W, D), jnp.float32),
            pltpu.VMEM((B_PER_W, D), jnp.float32),
            pltpu.VMEM((B_PER_W, D), jnp.float32),
            pltpu.VMEM_SHARED((NS * B_PER_W, D), jnp.float32),
            [pltpu.SemaphoreType.DMA] * N_CH,
            pltpu.SemaphoreType.DMA,
            pltpu.SemaphoreType.DMA,
        ],
    )
    def k(feat_hbm, nodes_hbm, nidx_hbm, self_out, mean_out,
          nidx_v, sidx_v, rows_v, self_v, mean_v, acc_sh,
          gsems, ssem, asem):
        sid = lax.axis_index("s")
        cid = lax.axis_index("c")
        wid = sid * NC + cid
        base = wid * B_PER_W
        sbase = sid * B_PER_W

        pltpu.sync_copy(nidx_hbm.at[pl.ds(wid * ROWS_PER_W, ROWS_PER_W)],
                        nidx_v)
        pltpu.sync_copy(nodes_hbm.at[pl.ds(base, B_PER_W)], sidx_v)

        gcopies = []
        for j in range(N_CH):
            gcopies.append(pltpu.async_copy(
                feat_hbm.at[nidx_v.at[pl.ds(j * GW, GW)]],
                rows_v.at[pl.ds(j * GW, GW)], gsems[j]))
        self_copy = pltpu.async_copy(feat_hbm.at[sidx_v], self_v, ssem)

        @pl.loop(0, B_PER_W)
        def _(n):
            for c in range(D // L):
                mean_v[n, pl.ds(c * L, L)] = jnp.zeros((L,), jnp.float32)

        pltpu.sync_copy(mean_v, acc_sh.at[pl.ds(sbase, B_PER_W)])

        acopies = []
        for j in range(N_CH):
            gcopies[j].wait()
            for n in range(j * NODES_PER_CH, (j + 1) * NODES_PER_CH):
                dst_rows = (sbase + n) + jnp.zeros((L,), jnp.int32)
                acopies.append(pltpu.async_copy(
                    rows_v.at[pl.ds(n * S, S)],
                    acc_sh.at[dst_rows], asem, add=True))
        for c in acopies:
            c.wait()

        pltpu.sync_copy(acc_sh.at[pl.ds(sbase, B_PER_W)], mean_v)

        @pl.loop(0, B_PER_W)
        def _(n):
            for c in range(D // L):
                cs = pl.ds(c * L, L)
                mean_v[n, cs] = mean_v[n, cs] * (1.0 / S)

        self_copy.wait()
        pltpu.sync_copy(self_v, self_out.at[pl.ds(base, B_PER_W)])
        pltpu.sync_copy(mean_v, mean_out.at[pl.ds(base, B_PER_W)])

    return k(features, nodes, neigh_flat)


def _tc_bmm(selff, meanf, weight):
    Bb = 64

    def body(s_ref, m_ref, w_ref, o_ref):
        c = jnp.concatenate([s_ref[...], m_ref[...]], axis=1)
        acc = jax.lax.dot_general(
            c, w_ref[...],
            dimension_numbers=(((1,), (1,)), ((0,), (0,))),
            preferred_element_type=jnp.float32)
        o_ref[...] = jnp.maximum(acc, 0.0)

    return pl.pallas_call(
        body,
        grid=(B // Bb,),
        in_specs=[
            pl.BlockSpec((Bb, D), lambda i: (i, 0)),
            pl.BlockSpec((Bb, D), lambda i: (i, 0)),
            pl.BlockSpec((Bb, 2 * D, E), lambda i: (i, 0, 0)),
        ],
        out_specs=pl.BlockSpec((Bb, E), lambda i: (i, 0)),
        out_shape=jax.ShapeDtypeStruct((B, E), jnp.float32),
    )(selff, meanf, weight)


def kernel(features, nodes, neigh_idx, weight):
    nodes = nodes.astype(jnp.int32)
    neigh_flat = neigh_idx.astype(jnp.int32).reshape(-1)
    selff, meanf = _sc_gather_mean(features, nodes, neigh_flat)
    return _tc_bmm(selff, meanf, weight)


# speedup vs baseline: 1.0685x; 1.0059x over previous
import functools

import jax
import jax.numpy as jnp
from jax import lax
from jax.experimental import pallas as pl
from jax.experimental.pallas import tpu as pltpu
from jax.experimental.pallas import tpu_sc as plsc

NC = 2
NS = 16
L = 16
NW = NC * NS

B = 1024
S = 16
D = 128
E = 128

B_PER_W = B // NW
ROWS_PER_W = B_PER_W * S
GW = 64
N_CH = ROWS_PER_W // GW
NODES_PER_CH = GW // S

_MESH = plsc.VectorSubcoreMesh(core_axis_name="c", subcore_axis_name="s")


def _sc_gather_mean(features, nodes, neigh_flat):
    @functools.partial(
        pl.kernel,
        out_type=(
            jax.ShapeDtypeStruct((B, D), jnp.float32),
            jax.ShapeDtypeStruct((B, D), jnp.float32),
        ),
        mesh=_MESH,
        scratch_types=[
            pltpu.VMEM((ROWS_PER_W,), jnp.int32),
            pltpu.VMEM((B_PER_W,), jnp.int32),
            pltpu.VMEM((ROWS_PER_W, D), jnp.float32),
            pltpu.VMEM((B_PER_W, D), jnp.float32),
            pltpu.VMEM((B_PER_W, D), jnp.float32),
            pltpu.VMEM_SHARED((NS * B_PER_W, D), jnp.float32),
            [pltpu.SemaphoreType.DMA] * N_CH,
            pltpu.SemaphoreType.DMA,
            pltpu.SemaphoreType.DMA,
        ],
    )
    def k(feat_hbm, nodes_hbm, nidx_hbm, self_out, mean_out,
          nidx_v, sidx_v, rows_v, self_v, mean_v, acc_sh,
          gsems, ssem, asem):
        sid = lax.axis_index("s")
        cid = lax.axis_index("c")
        wid = sid * NC + cid
        base = wid * B_PER_W
        sbase = sid * B_PER_W

        pltpu.sync_copy(nidx_hbm.at[pl.ds(wid * ROWS_PER_W, ROWS_PER_W)],
                        nidx_v)
        pltpu.sync_copy(nodes_hbm.at[pl.ds(base, B_PER_W)], sidx_v)

        gcopies = []
        for j in range(N_CH):
            gcopies.append(pltpu.async_copy(
                feat_hbm.at[nidx_v.at[pl.ds(j * GW, GW)]],
                rows_v.at[pl.ds(j * GW, GW)], gsems[j]))
        self_copy = pltpu.async_copy(feat_hbm.at[sidx_v], self_v, ssem)

        @pl.loop(0, B_PER_W)
        def _(n):
            for c in range(D // L):
                mean_v[n, pl.ds(c * L, L)] = jnp.zeros((L,), jnp.float32)

        pltpu.sync_copy(mean_v, acc_sh.at[pl.ds(sbase, B_PER_W)])

        acopies = []
        for j in range(N_CH):
            gcopies[j].wait()
            for n in range(j * NODES_PER_CH, (j + 1) * NODES_PER_CH):
                dst_rows = (sbase + n) + jnp.zeros((L,), jnp.int32)
                acopies.append(pltpu.async_copy(
                    rows_v.at[pl.ds(n * S, S)],
                    acc_sh.at[dst_rows], asem, add=True))
        for c in acopies:
            c.wait()

        pltpu.sync_copy(acc_sh.at[pl.ds(sbase, B_PER_W)], mean_v)

        @pl.loop(0, B_PER_W)
        def _(n):
            for c in range(D // L):
                cs = pl.ds(c * L, L)
                mean_v[n, cs] = mean_v[n, cs] * (1.0 / S)

        self_copy.wait()
        pltpu.sync_copy(self_v, self_out.at[pl.ds(base, B_PER_W)])
        pltpu.sync_copy(mean_v, mean_out.at[pl.ds(base, B_PER_W)])

    return k(features, nodes, neigh_flat)


def _tc_bmm(selff, meanf, weight):
    Bb = 64

    def body(s_ref, m_ref, w_ref, o_ref):
        c = jnp.concatenate([s_ref[...], m_ref[...]], axis=1)
        acc = jax.lax.dot_general(
            c, w_ref[...],
            dimension_numbers=(((1,), (1,)), ((0,), (0,))),
            preferred_element_type=jnp.float32)
        o_ref[...] = jnp.maximum(acc, 0.0)

    return pl.pallas_call(
        body,
        grid=(B // Bb,),
        in_specs=[
            pl.BlockSpec((Bb, D), lambda i: (i, 0)),
            pl.BlockSpec((Bb, D), lambda i: (i, 0)),
            pl.BlockSpec((Bb, 2 * D, E), lambda i: (i, 0, 0)),
        ],
        out_specs=pl.BlockSpec((Bb, E), lambda i: (i, 0)),
        out_shape=jax.ShapeDtypeStruct((B, E), jnp.float32),
    )(selff, meanf, weight)


def kernel(features, nodes, neigh_idx, weight):
    nodes = nodes.astype(jnp.int32)
    neigh_flat = neigh_idx.astype(jnp.int32).reshape(-1)
    selff, meanf = _sc_gather_mean(features, nodes, neigh_flat)
    return _tc_bmm(selff, meanf, weight)


# speedup vs baseline: 1.0782x; 1.0091x over previous
import functools

import jax
import jax.numpy as jnp
from jax import lax
from jax.experimental import pallas as pl
from jax.experimental.pallas import tpu as pltpu
from jax.experimental.pallas import tpu_sc as plsc

NC = 2
NS = 16
L = 16
NW = NC * NS

B = 1024
S = 16
D = 128
E = 128

B_PER_W = B // NW
ROWS_PER_W = B_PER_W * S
GW = 64
N_CH = ROWS_PER_W // GW
NODES_PER_CH = GW // S

_MESH = plsc.VectorSubcoreMesh(core_axis_name="c", subcore_axis_name="s")


def _sc_gather_mean(features, nodes, neigh_flat):
    @functools.partial(
        pl.kernel,
        out_type=(
            jax.ShapeDtypeStruct((B, D), jnp.float32),
            jax.ShapeDtypeStruct((B, D), jnp.float32),
        ),
        mesh=_MESH,
        scratch_types=[
            pltpu.VMEM((ROWS_PER_W,), jnp.int32),
            pltpu.VMEM((B_PER_W,), jnp.int32),
            pltpu.VMEM((ROWS_PER_W, D), jnp.float32),
            pltpu.VMEM((B_PER_W, D), jnp.float32),
            pltpu.VMEM((B_PER_W, D), jnp.float32),
            pltpu.VMEM_SHARED((NS * B_PER_W, D), jnp.float32),
            [pltpu.SemaphoreType.DMA] * N_CH,
            pltpu.SemaphoreType.DMA,
            pltpu.SemaphoreType.DMA,
        ],
    )
    def k(feat_hbm, nodes_hbm, nidx_hbm, self_out, mean_out,
          nidx_v, sidx_v, rows_v, self_v, mean_v, acc_sh,
          gsems, ssem, asem):
        sid = lax.axis_index("s")
        cid = lax.axis_index("c")
        wid = sid * NC + cid
        base = wid * B_PER_W
        sbase = sid * B_PER_W

        pltpu.sync_copy(nidx_hbm.at[pl.ds(wid * ROWS_PER_W, ROWS_PER_W)],
                        nidx_v)
        pltpu.sync_copy(nodes_hbm.at[pl.ds(base, B_PER_W)], sidx_v)

        gcopies = []
        for j in range(N_CH):
            gcopies.append(pltpu.async_copy(
                feat_hbm.at[nidx_v.at[pl.ds(j * GW, GW)]],
                rows_v.at[pl.ds(j * GW, GW)], gsems[j]))
        self_copy = pltpu.async_copy(feat_hbm.at[sidx_v], self_v, ssem)

        @pl.loop(0, B_PER_W)
        def _(n):
            for c in range(D // L):
                mean_v[n, pl.ds(c * L, L)] = jnp.zeros((L,), jnp.float32)

        pltpu.sync_copy(mean_v, acc_sh.at[pl.ds(sbase, B_PER_W)])

        acopies = []
        for j in range(N_CH):
            gcopies[j].wait()
            for n in range(j * NODES_PER_CH, (j + 1) * NODES_PER_CH):
                dst_rows = (sbase + n) + jnp.zeros((L,), jnp.int32)
                acopies.append(pltpu.async_copy(
                    rows_v.at[pl.ds(n * S, S)],
                    acc_sh.at[dst_rows], asem, add=True))
        for c in acopies:
            c.wait()

        self_copy.wait()
        pltpu.sync_copy(self_v, self_out.at[pl.ds(base, B_PER_W)])
        pltpu.sync_copy(acc_sh.at[pl.ds(sbase, B_PER_W)],
                        mean_out.at[pl.ds(base, B_PER_W)])

    return k(features, nodes, neigh_flat)


def _tc_bmm(selff, meanf, weight):
    Bb = 64

    def body(s_ref, m_ref, w_ref, o_ref):
        c = jnp.concatenate([s_ref[...], m_ref[...] * (1.0 / S)], axis=1)
        acc = jax.lax.dot_general(
            c, w_ref[...],
            dimension_numbers=(((1,), (1,)), ((0,), (0,))),
            preferred_element_type=jnp.float32)
        o_ref[...] = jnp.maximum(acc, 0.0)

    return pl.pallas_call(
        body,
        grid=(B // Bb,),
        in_specs=[
            pl.BlockSpec((Bb, D), lambda i: (i, 0)),
            pl.BlockSpec((Bb, D), lambda i: (i, 0)),
            pl.BlockSpec((Bb, 2 * D, E), lambda i: (i, 0, 0)),
        ],
        out_specs=pl.BlockSpec((Bb, E), lambda i: (i, 0)),
        out_shape=jax.ShapeDtypeStruct((B, E), jnp.float32),
    )(selff, meanf, weight)


def kernel(features, nodes, neigh_idx, weight):
    nodes = nodes.astype(jnp.int32)
    neigh_flat = neigh_idx.astype(jnp.int32).reshape(-1)
    selff, meanf = _sc_gather_mean(features, nodes, neigh_flat)
    return _tc_bmm(selff, meanf, weight)


# speedup vs baseline: 1.0831x; 1.0046x over previous
import functools

import jax
import jax.numpy as jnp
from jax import lax
from jax.experimental import pallas as pl
from jax.experimental.pallas import tpu as pltpu
from jax.experimental.pallas import tpu_sc as plsc

NC = 2
NS = 16
L = 16
NW = NC * NS

B = 1024
S = 16
D = 128
E = 128

B_PER_W = B // NW
ROWS_PER_W = B_PER_W * S
GW = 32
N_CH = ROWS_PER_W // GW
NODES_PER_CH = GW // S

_MESH = plsc.VectorSubcoreMesh(core_axis_name="c", subcore_axis_name="s")


def _sc_gather_mean(features, nodes, neigh_flat):
    @functools.partial(
        pl.kernel,
        out_type=(
            jax.ShapeDtypeStruct((B, D), jnp.float32),
            jax.ShapeDtypeStruct((B, D), jnp.float32),
        ),
        mesh=_MESH,
        scratch_types=[
            pltpu.VMEM((ROWS_PER_W,), jnp.int32),
            pltpu.VMEM((B_PER_W,), jnp.int32),
            pltpu.VMEM((ROWS_PER_W, D), jnp.float32),
            pltpu.VMEM((B_PER_W, D), jnp.float32),
            pltpu.VMEM((B_PER_W, D), jnp.float32),
            pltpu.VMEM_SHARED((NS * B_PER_W, D), jnp.float32),
            [pltpu.SemaphoreType.DMA] * N_CH,
            pltpu.SemaphoreType.DMA,
            pltpu.SemaphoreType.DMA,
        ],
    )
    def k(feat_hbm, nodes_hbm, nidx_hbm, self_out, mean_out,
          nidx_v, sidx_v, rows_v, self_v, mean_v, acc_sh,
          gsems, ssem, asem):
        sid = lax.axis_index("s")
        cid = lax.axis_index("c")
        wid = sid * NC + cid
        base = wid * B_PER_W
        sbase = sid * B_PER_W

        pltpu.sync_copy(nidx_hbm.at[pl.ds(wid * ROWS_PER_W, ROWS_PER_W)],
                        nidx_v)
        pltpu.sync_copy(nodes_hbm.at[pl.ds(base, B_PER_W)], sidx_v)

        gcopies = []
        for j in range(N_CH):
            gcopies.append(pltpu.async_copy(
                feat_hbm.at[nidx_v.at[pl.ds(j * GW, GW)]],
                rows_v.at[pl.ds(j * GW, GW)], gsems[j]))
        self_copy = pltpu.async_copy(feat_hbm.at[sidx_v], self_v, ssem)

        @pl.loop(0, B_PER_W)
        def _(n):
            for c in range(D // L):
                mean_v[n, pl.ds(c * L, L)] = jnp.zeros((L,), jnp.float32)

        pltpu.sync_copy(mean_v, acc_sh.at[pl.ds(sbase, B_PER_W)])

        acopies = []
        for j in range(N_CH):
            gcopies[j].wait()
            for n in range(j * NODES_PER_CH, (j + 1) * NODES_PER_CH):
                dst_rows = (sbase + n) + jnp.zeros((L,), jnp.int32)
                acopies.append(pltpu.async_copy(
                    rows_v.at[pl.ds(n * S, S)],
                    acc_sh.at[dst_rows], asem, add=True))
        for c in acopies:
            c.wait()

        self_copy.wait()
        pltpu.sync_copy(self_v, self_out.at[pl.ds(base, B_PER_W)])
        pltpu.sync_copy(acc_sh.at[pl.ds(sbase, B_PER_W)],
                        mean_out.at[pl.ds(base, B_PER_W)])

    return k(features, nodes, neigh_flat)


def _tc_bmm(selff, meanf, weight):
    Bb = 64

    def body(s_ref, m_ref, w_ref, o_ref):
        c = jnp.concatenate([s_ref[...], m_ref[...] * (1.0 / S)], axis=1)
        acc = jax.lax.dot_general(
            c, w_ref[...],
            dimension_numbers=(((1,), (1,)), ((0,), (0,))),
            preferred_element_type=jnp.float32)
        o_ref[...] = jnp.maximum(acc, 0.0)

    return pl.pallas_call(
        body,
        grid=(B // Bb,),
        in_specs=[
            pl.BlockSpec((Bb, D), lambda i: (i, 0)),
            pl.BlockSpec((Bb, D), lambda i: (i, 0)),
            pl.BlockSpec((Bb, 2 * D, E), lambda i: (i, 0, 0)),
        ],
        out_specs=pl.BlockSpec((Bb, E), lambda i: (i, 0)),
        out_shape=jax.ShapeDtypeStruct((B, E), jnp.float32),
    )(selff, meanf, weight)


def kernel(features, nodes, neigh_idx, weight):
    nodes = nodes.astype(jnp.int32)
    neigh_flat = neigh_idx.astype(jnp.int32).reshape(-1)
    selff, meanf = _sc_gather_mean(features, nodes, neigh_flat)
    return _tc_bmm(selff, meanf, weight)
